# baseline (speedup 1.0000x reference)
import functools

import jax
import jax.numpy as jnp
from jax import lax
from jax.experimental import pallas as pl
from jax.experimental.pallas import tpu as pltpu
from jax.experimental.pallas import tpu_sc as plsc

N_NODES = 10000
N_EDGES = 320000
D = 128

NC = 2
NS = 16
NW = NC * NS

CH = 128
WIN = 16
NWIN = 5
NCH = WIN * NWIN
EPT = NCH * CH
EPAD = NW * EPT
TRASH = 10239

NPAD = 10240
RPT = NPAD // NS
RCOPIES = RPT // CH


def _sc_body(x_ref, src_ref, dst_ref, ones_ref, agg_out, cnt_out,
             idxs_v, idxd_v, buf_a, buf_b, acc_sh, sem_a, sem_b):
    cid = lax.axis_index("c")
    sid = lax.axis_index("s")
    wid = cid * NS + sid

    zeros16 = jnp.zeros((16,), jnp.float32)
    one16 = jnp.full((16,), 1.0, jnp.float32)

    def fill_zero(i, carry):
        for c in range(D // 16):
            buf_b[i, pl.ds(c * 16, 16)] = zeros16
        return carry

    lax.fori_loop(0, CH, fill_zero, 0)

    def zero_acc(j, carry):
        pltpu.sync_copy(buf_b, acc_sh.at[pl.ds(sid * RPT + j * CH, CH), :])
        return carry

    lax.fori_loop(0, RCOPIES, zero_acc, 0)
    plsc.subcore_barrier()

    def window1(w, carry):
        pltpu.sync_copy(src_ref.at[wid, pl.ds(w * WIN, WIN)], idxs_v)
        pltpu.sync_copy(dst_ref.at[wid, pl.ds(w * WIN, WIN)], idxd_v)

        def chunk(j, c2):
            pltpu.async_copy(x_ref.at[idxs_v.at[j]], buf_a, sem_a).wait()
            pltpu.sync_copy(buf_a, acc_sh.at[idxd_v.at[j]], add=True)
            return c2

        lax.fori_loop(0, WIN, chunk, 0)
        return carry

    lax.fori_loop(0, NWIN, window1, 0)
    plsc.subcore_barrier()

    def out_copy1(j, carry):
        base = sid * RPT + j * CH
        pltpu.sync_copy(acc_sh.at[pl.ds(base, CH), :],
                        agg_out.at[cid, pl.ds(base, CH), :])
        return carry

    lax.fori_loop(0, RCOPIES, out_copy1, 0)

    lax.fori_loop(0, CH, fill_zero, 0)
    lax.fori_loop(0, RCOPIES, zero_acc, 0)
    pltpu.sync_copy(ones_ref, buf_a)
    plsc.subcore_barrier()

    def window2(w, carry):
        pltpu.sync_copy(dst_ref.at[wid, pl.ds(w * WIN, WIN)], idxd_v)

        def cchunk(i, c2):
            pltpu.sync_copy(buf_a, acc_sh.at[idxd_v.at[i]], add=True)
            return c2

        lax.fori_loop(0, WIN, cchunk, 0)
        return carry

    lax.fori_loop(0, NWIN, window2, 0)
    plsc.subcore_barrier()

    def out_copy2(j, carry):
        base = sid * RPT + j * CH
        pltpu.sync_copy(acc_sh.at[pl.ds(base, CH), :],
                        cnt_out.at[cid, pl.ds(base, CH), :])
        return carry

    lax.fori_loop(0, RCOPIES, out_copy2, 0)


@functools.lru_cache(maxsize=1)
def _sc_agg():
    return functools.partial(
        pl.kernel,
        mesh=plsc.VectorSubcoreMesh(core_axis_name="c", subcore_axis_name="s",
                                    num_cores=NC, num_subcores=NS),
        out_type=(
            jax.ShapeDtypeStruct((NC, NPAD, D), jnp.float32),
            jax.ShapeDtypeStruct((NC, NPAD, D), jnp.float32),
        ),
        scratch_types=[
            pltpu.VMEM((WIN, CH), jnp.int32),
            pltpu.VMEM((WIN, CH), jnp.int32),
            pltpu.VMEM((CH, D), jnp.float32),
            pltpu.VMEM((CH, D), jnp.float32),
            pltpu.VMEM_SHARED((NPAD, D), jnp.float32),
            pltpu.SemaphoreType.DMA,
            pltpu.SemaphoreType.DMA,
        ],
    )(_sc_body)


BM = 1000


def _tc_body(p_ref, c_ref, x_ref, w_ref, b_ref, o_ref):
    pa = p_ref[0] + p_ref[1]
    cnt = c_ref[0, :, 0:1] + c_ref[1, :, 0:1]
    inv = jnp.where(cnt > 0, 1.0 / cnt, 0.0)
    h = pa * inv
    dn = (((1,), (1,)), ((), ()))
    out = lax.dot_general(h, w_ref[:, 0:D], dn,
                          preferred_element_type=jnp.float32,
                          precision=lax.Precision.HIGHEST)
    out += lax.dot_general(x_ref[...], w_ref[:, D:2 * D], dn,
                           preferred_element_type=jnp.float32,
                           precision=lax.Precision.HIGHEST)
    o_ref[...] = out + b_ref[...]


def _tc_linear(p, c, x, W, b2):
    return pl.pallas_call(
        _tc_body,
        grid=(N_NODES // BM,),
        in_specs=[
            pl.BlockSpec((NC, BM, D), lambda m: (0, m, 0)),
            pl.BlockSpec((NC, BM, D), lambda m: (0, m, 0)),
            pl.BlockSpec((BM, D), lambda m: (m, 0)),
            pl.BlockSpec((D, 2 * D), lambda m: (0, 0)),
            pl.BlockSpec((1, D), lambda m: (0, 0)),
        ],
        out_specs=pl.BlockSpec((BM, D), lambda m: (m, 0)),
        out_shape=jax.ShapeDtypeStruct((N_NODES, D), jnp.float32),
    )(p, c, x, W, b2)


def kernel(x, edge_index, W, b):
    src = edge_index[0].astype(jnp.int32)
    dst = edge_index[1].astype(jnp.int32)
    pad = EPAD - N_EDGES
    srcp = jnp.concatenate([src, jnp.zeros((pad,), jnp.int32)]
                           ).reshape(NW, NCH, CH)
    dstp = jnp.concatenate([dst, jnp.full((pad,), TRASH, jnp.int32)]
                           ).reshape(NW, NCH, CH)
    onesrow = jnp.zeros((CH, D), jnp.float32).at[:, 0].set(1.0)
    p, c = _sc_agg()(x, srcp, dstp, onesrow)
    b2 = b.reshape(1, D)
    return _tc_linear(p, c, x, W, b2)


# speedup vs baseline: 2.7187x; 2.7187x over previous
import functools

import jax
import jax.numpy as jnp
from jax import lax
from jax.experimental import pallas as pl
from jax.experimental.pallas import tpu as pltpu
from jax.experimental.pallas import tpu_sc as plsc

N_NODES = 10000
N_EDGES = 320000
D = 128

NC = 2
NS = 16
NW = NC * NS

EPW = N_EDGES // NW
CH = 80
NCHUNK = EPW // CH
NPAIR = NCHUNK // 2

NPAD = 10240
RPT = NPAD // NS
RCOPIES = RPT // CH


def _sc_body(x_ref, src_ref, dst_ref, ones_ref, agg_out, cnt_out,
             src_a, src_b, dst_all, buf_a, buf_b, acc_sh, sem_a, sem_b):
    cid = lax.axis_index("c")
    sid = lax.axis_index("s")
    wid = cid * NS + sid

    zeros16 = jnp.zeros((16,), jnp.float32)

    def fill_zero(i, carry):
        for c in range(D // 16):
            buf_b[i, pl.ds(c * 16, 16)] = zeros16
        return carry

    lax.fori_loop(0, CH, fill_zero, 0)

    def zero_acc(j, carry):
        pltpu.sync_copy(buf_b, acc_sh.at[pl.ds(sid * RPT + j * CH, CH), :])
        return carry

    lax.fori_loop(0, RCOPIES, zero_acc, 0)
    pltpu.sync_copy(dst_ref.at[wid], dst_all)
    plsc.subcore_barrier()

    pltpu.sync_copy(src_ref.at[wid, 0], src_a)
    pltpu.async_copy(x_ref.at[src_a], buf_a, sem_a)

    def pair(j, carry):
        c0 = 2 * j
        c1 = 2 * j + 1
        pltpu.sync_copy(src_ref.at[wid, c1], src_b)
        pltpu.async_copy(x_ref.at[src_b], buf_b, sem_b)
        pltpu.make_async_copy(x_ref.at[src_a], buf_a, sem_a).wait()
        pltpu.sync_copy(buf_a, acc_sh.at[dst_all.at[c0]], add=True)
        pltpu.sync_copy(src_ref.at[wid, c1 + 1], src_a)
        pltpu.async_copy(x_ref.at[src_a], buf_a, sem_a)
        pltpu.make_async_copy(x_ref.at[src_b], buf_b, sem_b).wait()
        pltpu.sync_copy(buf_b, acc_sh.at[dst_all.at[c1]], add=True)
        return carry

    lax.fori_loop(0, NPAIR, pair, 0)
    pltpu.make_async_copy(x_ref.at[src_a], buf_a, sem_a).wait()
    pltpu.sync_copy(buf_a, acc_sh.at[dst_all.at[NCHUNK - 1]], add=True)

    plsc.subcore_barrier()

    def out_copy1(j, carry):
        base = sid * RPT + j * CH
        pltpu.sync_copy(acc_sh.at[pl.ds(base, CH), :],
                        agg_out.at[cid, pl.ds(base, CH), :])
        return carry

    lax.fori_loop(0, RCOPIES, out_copy1, 0)

    lax.fori_loop(0, CH, fill_zero, 0)
    lax.fori_loop(0, RCOPIES, zero_acc, 0)
    pltpu.sync_copy(ones_ref, buf_a)
    plsc.subcore_barrier()

    def chunk2(i, carry):
        pltpu.sync_copy(buf_a, acc_sh.at[dst_all.at[i]], add=True)
        return carry

    lax.fori_loop(0, NCHUNK, chunk2, 0)
    plsc.subcore_barrier()

    def out_copy2(j, carry):
        base = sid * RPT + j * CH
        pltpu.sync_copy(acc_sh.at[pl.ds(base, CH), :],
                        cnt_out.at[cid, pl.ds(base, CH), :])
        return carry

    lax.fori_loop(0, RCOPIES, out_copy2, 0)


@functools.lru_cache(maxsize=1)
def _sc_agg():
    return functools.partial(
        pl.kernel,
        mesh=plsc.VectorSubcoreMesh(core_axis_name="c", subcore_axis_name="s",
                                    num_cores=NC, num_subcores=NS),
        out_type=(
            jax.ShapeDtypeStruct((NC, NPAD, D), jnp.float32),
            jax.ShapeDtypeStruct((NC, NPAD, D), jnp.float32),
        ),
        scratch_types=[
            pltpu.VMEM((CH,), jnp.int32),
            pltpu.VMEM((CH,), jnp.int32),
            pltpu.VMEM((NCHUNK, CH), jnp.int32),
            pltpu.VMEM((CH, D), jnp.float32),
            pltpu.VMEM((CH, D), jnp.float32),
            pltpu.VMEM_SHARED((NPAD, D), jnp.float32),
            pltpu.SemaphoreType.DMA,
            pltpu.SemaphoreType.DMA,
        ],
    )(_sc_body)


BM = 1000


def _tc_body(p_ref, c_ref, x_ref, w_ref, b_ref, o_ref):
    pa = p_ref[0] + p_ref[1]
    cnt = c_ref[0, :, 0:1] + c_ref[1, :, 0:1]
    inv = jnp.where(cnt > 0, 1.0 / cnt, 0.0)
    h = pa * inv
    dn = (((1,), (1,)), ((), ()))
    out = lax.dot_general(h, w_ref[:, 0:D], dn,
                          preferred_element_type=jnp.float32,
                          precision=lax.Precision.HIGHEST)
    out += lax.dot_general(x_ref[...], w_ref[:, D:2 * D], dn,
                           preferred_element_type=jnp.float32,
                           precision=lax.Precision.HIGHEST)
    o_ref[...] = out + b_ref[...]


def _tc_linear(p, c, x, W, b2):
    return pl.pallas_call(
        _tc_body,
        grid=(N_NODES // BM,),
        in_specs=[
            pl.BlockSpec((NC, BM, D), lambda m: (0, m, 0)),
            pl.BlockSpec((NC, BM, D), lambda m: (0, m, 0)),
            pl.BlockSpec((BM, D), lambda m: (m, 0)),
            pl.BlockSpec((D, 2 * D), lambda m: (0, 0)),
            pl.BlockSpec((1, D), lambda m: (0, 0)),
        ],
        out_specs=pl.BlockSpec((BM, D), lambda m: (m, 0)),
        out_shape=jax.ShapeDtypeStruct((N_NODES, D), jnp.float32),
    )(p, c, x, W, b2)


def kernel(x, edge_index, W, b):
    src = edge_index[0].astype(jnp.int32).reshape(NW, NCHUNK, CH)
    dst = edge_index[1].astype(jnp.int32).reshape(NW, NCHUNK, CH)
    onesrow = jnp.zeros((CH, D), jnp.float32).at[:, 0].set(1.0)
    p, c = _sc_agg()(x, src, dst, onesrow)
    b2 = b.reshape(1, D)
    return _tc_linear(p, c, x, W, b2)


# speedup vs baseline: 2.7296x; 1.0040x over previous
import functools

import jax
import jax.numpy as jnp
from jax import lax
from jax.experimental import pallas as pl
from jax.experimental.pallas import tpu as pltpu
from jax.experimental.pallas import tpu_sc as plsc

N_NODES = 10000
N_EDGES = 320000
D = 128

NC = 2
NS = 16
NW = NC * NS

EPW = N_EDGES // NW
CH = 80
NCHUNK = EPW // CH
NPAIR = NCHUNK // 2

NPAD = 10240
RPT = NPAD // NS
RCOPIES = RPT // CH


def _sc_body(x_ref, src_ref, dst_ref, ones_ref, agg_out, cnt_out,
             src_a, src_b, dst_all, buf_a, buf_b, acc_sh, sem_a, sem_b):
    cid = lax.axis_index("c")
    sid = lax.axis_index("s")
    wid = cid * NS + sid

    zeros16 = jnp.zeros((16,), jnp.float32)

    def fill_zero(i, carry):
        for c in range(D // 16):
            buf_b[i, pl.ds(c * 16, 16)] = zeros16
        return carry

    lax.fori_loop(0, CH, fill_zero, 0)

    def zero_acc(j, carry):
        pltpu.sync_copy(buf_b, acc_sh.at[pl.ds(sid * RPT + j * CH, CH), :])
        return carry

    lax.fori_loop(0, RCOPIES, zero_acc, 0)
    pltpu.sync_copy(dst_ref.at[wid], dst_all)
    plsc.subcore_barrier()

    pltpu.sync_copy(src_ref.at[wid, 0], src_a)
    pltpu.async_copy(x_ref.at[src_a], buf_a, sem_a)

    def pair(j, carry):
        c0 = 2 * j
        c1 = 2 * j + 1
        pltpu.sync_copy(src_ref.at[wid, c1], src_b)
        pltpu.async_copy(x_ref.at[src_b], buf_b, sem_b)
        pltpu.make_async_copy(x_ref.at[src_a], buf_a, sem_a).wait()
        pltpu.sync_copy(buf_a, acc_sh.at[dst_all.at[c0]], add=True)
        pltpu.sync_copy(src_ref.at[wid, c1 + 1], src_a)
        pltpu.async_copy(x_ref.at[src_a], buf_a, sem_a)
        pltpu.make_async_copy(x_ref.at[src_b], buf_b, sem_b).wait()
        pltpu.sync_copy(buf_b, acc_sh.at[dst_all.at[c1]], add=True)
        return carry

    lax.fori_loop(0, NPAIR, pair, 0)
    pltpu.make_async_copy(x_ref.at[src_a], buf_a, sem_a).wait()
    pltpu.sync_copy(buf_a, acc_sh.at[dst_all.at[NCHUNK - 1]], add=True)

    plsc.subcore_barrier()

    def out_copy1(j, carry):
        base = sid * RPT + j * CH
        pltpu.sync_copy(acc_sh.at[pl.ds(base, CH), :],
                        agg_out.at[cid, pl.ds(base, CH), :])
        return carry

    lax.fori_loop(0, RCOPIES, out_copy1, 0)

    lax.fori_loop(0, CH, fill_zero, 0)
    lax.fori_loop(0, RCOPIES, zero_acc, 0)
    pltpu.sync_copy(ones_ref, buf_a)
    plsc.subcore_barrier()

    K2 = 5

    def chunk2(g, carry):
        for u in range(K2):
            pltpu.async_copy(buf_a, acc_sh.at[dst_all.at[g * K2 + u]], sem_a,
                             add=True)
        for u in range(K2):
            pltpu.make_async_copy(buf_a, acc_sh.at[dst_all.at[0]],
                                  sem_a).wait()
        return carry

    lax.fori_loop(0, NCHUNK // K2, chunk2, 0)
    plsc.subcore_barrier()

    def out_copy2(j, carry):
        base = sid * RPT + j * CH
        pltpu.sync_copy(acc_sh.at[pl.ds(base, CH), :],
                        cnt_out.at[cid, pl.ds(base, CH), :])
        return carry

    lax.fori_loop(0, RCOPIES, out_copy2, 0)


@functools.lru_cache(maxsize=1)
def _sc_agg():
    return functools.partial(
        pl.kernel,
        mesh=plsc.VectorSubcoreMesh(core_axis_name="c", subcore_axis_name="s",
                                    num_cores=NC, num_subcores=NS),
        out_type=(
            jax.ShapeDtypeStruct((NC, NPAD, D), jnp.float32),
            jax.ShapeDtypeStruct((NC, NPAD, D), jnp.float32),
        ),
        scratch_types=[
            pltpu.VMEM((CH,), jnp.int32),
            pltpu.VMEM((CH,), jnp.int32),
            pltpu.VMEM((NCHUNK, CH), jnp.int32),
            pltpu.VMEM((CH, D), jnp.float32),
            pltpu.VMEM((CH, D), jnp.float32),
            pltpu.VMEM_SHARED((NPAD, D), jnp.float32),
            pltpu.SemaphoreType.DMA,
            pltpu.SemaphoreType.DMA,
        ],
    )(_sc_body)


BM = 1000


def _tc_body(p_ref, c_ref, x_ref, w_ref, b_ref, o_ref):
    pa = p_ref[0] + p_ref[1]
    cnt = c_ref[0, :, 0:1] + c_ref[1, :, 0:1]
    inv = jnp.where(cnt > 0, 1.0 / cnt, 0.0)
    h = pa * inv
    dn = (((1,), (1,)), ((), ()))
    out = lax.dot_general(h, w_ref[:, 0:D], dn,
                          preferred_element_type=jnp.float32,
                          precision=lax.Precision.HIGHEST)
    out += lax.dot_general(x_ref[...], w_ref[:, D:2 * D], dn,
                           preferred_element_type=jnp.float32,
                           precision=lax.Precision.HIGHEST)
    o_ref[...] = out + b_ref[...]


def _tc_linear(p, c, x, W, b2):
    return pl.pallas_call(
        _tc_body,
        grid=(N_NODES // BM,),
        in_specs=[
            pl.BlockSpec((NC, BM, D), lambda m: (0, m, 0)),
            pl.BlockSpec((NC, BM, D), lambda m: (0, m, 0)),
            pl.BlockSpec((BM, D), lambda m: (m, 0)),
            pl.BlockSpec((D, 2 * D), lambda m: (0, 0)),
            pl.BlockSpec((1, D), lambda m: (0, 0)),
        ],
        out_specs=pl.BlockSpec((BM, D), lambda m: (m, 0)),
        out_shape=jax.ShapeDtypeStruct((N_NODES, D), jnp.float32),
    )(p, c, x, W, b2)


def kernel(x, edge_index, W, b):
    src = edge_index[0].astype(jnp.int32).reshape(NW, NCHUNK, CH)
    dst = edge_index[1].astype(jnp.int32).reshape(NW, NCHUNK, CH)
    onesrow = jnp.zeros((CH, D), jnp.float32).at[:, 0].set(1.0)
    p, c = _sc_agg()(x, src, dst, onesrow)
    b2 = b.reshape(1, D)
    return _tc_linear(p, c, x, W, b2)


# speedup vs baseline: 2.8052x; 1.0277x over previous
import functools

import jax
import jax.numpy as jnp
from jax import lax
from jax.experimental import pallas as pl
from jax.experimental.pallas import tpu as pltpu
from jax.experimental.pallas import tpu_sc as plsc

N_NODES = 10000
N_EDGES = 320000
D = 128

NC = 2
NS = 16
NW = NC * NS

EPW = N_EDGES // NW
CH = 80
NCHUNK = EPW // CH
NPAIR = NCHUNK // 2

NPAD = 10240
RPT = NPAD // NS
RCOPIES = RPT // CH


def _sc_body(x_ref, src_ref, dst_ref, ones_ref, agg_out, cnt_out,
             src_a, src_b, dst_all, buf_a, buf_b, acc_sh, sem_a, sem_b):
    cid = lax.axis_index("c")
    sid = lax.axis_index("s")
    wid = cid * NS + sid

    zeros16 = jnp.zeros((16,), jnp.float32)

    def fill_zero(i, carry):
        for c in range(D // 16):
            buf_b[i, pl.ds(c * 16, 16)] = zeros16
        return carry

    lax.fori_loop(0, CH, fill_zero, 0)

    def zero_acc(j, carry):
        pltpu.sync_copy(buf_b, acc_sh.at[pl.ds(sid * RPT + j * CH, CH), :])
        return carry

    lax.fori_loop(0, RCOPIES, zero_acc, 0)
    pltpu.sync_copy(dst_ref.at[wid], dst_all)
    plsc.subcore_barrier()

    pltpu.sync_copy(src_ref.at[wid, 0], src_a)
    pltpu.async_copy(x_ref.at[src_a], buf_a, sem_a)

    def pair(j, carry):
        c0 = 2 * j
        c1 = 2 * j + 1
        pltpu.sync_copy(src_ref.at[wid, c1], src_b)
        pltpu.async_copy(x_ref.at[src_b], buf_b, sem_b)
        pltpu.make_async_copy(x_ref.at[src_a], buf_a, sem_a).wait()
        pltpu.sync_copy(buf_a, acc_sh.at[dst_all.at[c0]], add=True)
        pltpu.sync_copy(src_ref.at[wid, c1 + 1], src_a)
        pltpu.async_copy(x_ref.at[src_a], buf_a, sem_a)
        pltpu.make_async_copy(x_ref.at[src_b], buf_b, sem_b).wait()
        pltpu.sync_copy(buf_b, acc_sh.at[dst_all.at[c1]], add=True)
        return carry

    lax.fori_loop(0, NPAIR, pair, 0)
    pltpu.make_async_copy(x_ref.at[src_a], buf_a, sem_a).wait()
    pltpu.sync_copy(buf_a, acc_sh.at[dst_all.at[NCHUNK - 1]], add=True)

    plsc.subcore_barrier()

    def out_copy1(j, carry):
        base = sid * RPT + j * CH
        pltpu.sync_copy(acc_sh.at[pl.ds(base, CH), :],
                        agg_out.at[cid, pl.ds(base, CH), :])
        return carry

    lax.fori_loop(0, RCOPIES, out_copy1, 0)

    lax.fori_loop(0, CH, fill_zero, 0)
    lax.fori_loop(0, RCOPIES, zero_acc, 0)
    pltpu.sync_copy(ones_ref, buf_a)
    plsc.subcore_barrier()

    K2 = 5

    def chunk2(g, carry):
        for u in range(K2):
            pltpu.async_copy(buf_a, acc_sh.at[dst_all.at[g * K2 + u]], sem_a,
                             add=True)
        for u in range(K2):
            pltpu.make_async_copy(buf_a, acc_sh.at[dst_all.at[0]],
                                  sem_a).wait()
        return carry

    lax.fori_loop(0, NCHUNK // K2, chunk2, 0)
    plsc.subcore_barrier()

    def out_copy2(j, carry):
        base = sid * RPT + j * CH
        pltpu.sync_copy(acc_sh.at[pl.ds(base, CH), :],
                        cnt_out.at[cid, pl.ds(base, CH), :])
        return carry

    lax.fori_loop(0, RCOPIES, out_copy2, 0)


@functools.lru_cache(maxsize=1)
def _sc_agg():
    return functools.partial(
        pl.kernel,
        mesh=plsc.VectorSubcoreMesh(core_axis_name="c", subcore_axis_name="s",
                                    num_cores=NC, num_subcores=NS),
        out_type=(
            jax.ShapeDtypeStruct((NC, NPAD, D), jnp.float32),
            jax.ShapeDtypeStruct((NC, NPAD, D), jnp.float32),
        ),
        scratch_types=[
            pltpu.VMEM((CH,), jnp.int32),
            pltpu.VMEM((CH,), jnp.int32),
            pltpu.VMEM((NCHUNK, CH), jnp.int32),
            pltpu.VMEM((CH, D), jnp.float32),
            pltpu.VMEM((CH, D), jnp.float32),
            pltpu.VMEM_SHARED((NPAD, D), jnp.float32),
            pltpu.SemaphoreType.DMA,
            pltpu.SemaphoreType.DMA,
        ],
    )(_sc_body)


BM = 1000


_DN = (((1,), (1,)), ((), ()))


def _tc_pre_body(x_ref, w_ref, b_ref, o_ref):
    o_ref[...] = lax.dot_general(x_ref[...], w_ref[:, D:2 * D], _DN,
                                 preferred_element_type=jnp.float32,
                                 precision=lax.Precision.DEFAULT) + b_ref[...]


def _tc_pre(x, W, b2):
    return pl.pallas_call(
        _tc_pre_body,
        grid=(N_NODES // BM,),
        in_specs=[
            pl.BlockSpec((BM, D), lambda m: (m, 0)),
            pl.BlockSpec((D, 2 * D), lambda m: (0, 0)),
            pl.BlockSpec((1, D), lambda m: (0, 0)),
        ],
        out_specs=pl.BlockSpec((BM, D), lambda m: (m, 0)),
        out_shape=jax.ShapeDtypeStruct((N_NODES, D), jnp.float32),
    )(x, W, b2)


def _tc_body(p_ref, c_ref, xw_ref, w_ref, o_ref):
    pa = p_ref[0] + p_ref[1]
    cnt = c_ref[0, :, 0:1] + c_ref[1, :, 0:1]
    inv = jnp.where(cnt > 0, 1.0 / cnt, 0.0)
    h = pa * inv
    o_ref[...] = lax.dot_general(h, w_ref[:, 0:D], _DN,
                                 preferred_element_type=jnp.float32,
                                 precision=lax.Precision.DEFAULT) + xw_ref[...]


def _tc_linear(p, c, xw, W):
    return pl.pallas_call(
        _tc_body,
        grid=(N_NODES // BM,),
        in_specs=[
            pl.BlockSpec((NC, BM, D), lambda m: (0, m, 0)),
            pl.BlockSpec((NC, BM, D), lambda m: (0, m, 0)),
            pl.BlockSpec((BM, D), lambda m: (m, 0)),
            pl.BlockSpec((D, 2 * D), lambda m: (0, 0)),
        ],
        out_specs=pl.BlockSpec((BM, D), lambda m: (m, 0)),
        out_shape=jax.ShapeDtypeStruct((N_NODES, D), jnp.float32),
    )(p, c, xw, W)


def kernel(x, edge_index, W, b):
    src = edge_index[0].astype(jnp.int32).reshape(NW, NCHUNK, CH)
    dst = edge_index[1].astype(jnp.int32).reshape(NW, NCHUNK, CH)
    onesrow = jnp.zeros((CH, D), jnp.float32).at[:, 0].set(1.0)
    b2 = b.reshape(1, D)
    xw = _tc_pre(x, W, b2)
    p, c = _sc_agg()(x, src, dst, onesrow)
    return _tc_linear(p, c, xw, W)
